# Initial kernel scaffold; baseline (speedup 1.0000x reference)
import jax
import jax.numpy as jnp
from jax.experimental import pallas as pl


def kernel(x, edge_index, W1, b1, W2, b2):
    raise NotImplementedError("write your pallas kernel here")


# speedup vs baseline: 14.9464x; 14.9464x over previous
import functools

import jax
import jax.numpy as jnp
from jax import lax
from jax.experimental import pallas as pl
from jax.experimental.pallas import tpu as pltpu
from jax.experimental.pallas import tpu_sc as plsc

N = 10000
NP = 10240
E = 320000
ER = E // 128
C = 128
RB = 1024


def _mesh():
    return plsc.VectorSubcoreMesh(core_axis_name="c", subcore_axis_name="s")


def _deg_kernel(dst2d):
    @functools.partial(
        pl.kernel,
        mesh=_mesh(),
        out_type=jax.ShapeDtypeStruct((2, NP, C), jnp.float32),
        scratch_types=[
            pltpu.VMEM_SHARED((NP, C), jnp.float32),
            pltpu.VMEM((128, C), jnp.float32),
            pltpu.VMEM((128,), jnp.int32),
        ],
    )
    def k(dst_hbm, out_hbm, acc, buf, didx):
        c = lax.axis_index("c")
        s = lax.axis_index("s")
        zero16 = jnp.zeros((16,), jnp.float32)
        ones16 = jnp.ones((16,), jnp.float32)

        def zb(i, _):
            buf[i // 8, pl.ds((i % 8) * 16, 16)] = zero16
            return 0

        lax.fori_loop(0, 1024, zb, 0)
        for j in range(5):
            pltpu.sync_copy(buf, acc.at[pl.ds(s * 640 + j * 128, 128)])

        def ob(i, _):
            buf[i // 8, pl.ds((i % 8) * 16, 16)] = ones16
            return 0

        lax.fori_loop(0, 1024, ob, 0)
        plsc.subcore_barrier()

        start = c * (ER // 2) + (s * (ER // 2)) // 16
        end = c * (ER // 2) + ((s + 1) * (ER // 2)) // 16

        def body(r, _):
            pltpu.sync_copy(dst_hbm.at[r], didx)
            pltpu.sync_copy(buf, acc.at[didx], add=True)
            return 0

        lax.fori_loop(start, end, body, 0)
        plsc.subcore_barrier()
        pltpu.sync_copy(acc.at[pl.ds(s * 640, 640)],
                        out_hbm.at[c, pl.ds(s * 640, 640)])

    return k(dst2d)


def _agg_kernel(hs, src2d, dst2d):
    @functools.partial(
        pl.kernel,
        mesh=_mesh(),
        out_type=jax.ShapeDtypeStruct((2, NP, C), jnp.float32),
        scratch_types=[
            pltpu.VMEM_SHARED((NP, C), jnp.float32),
            pltpu.VMEM((128, C), jnp.float32),
            pltpu.VMEM((128,), jnp.int32),
            pltpu.VMEM((128,), jnp.int32),
            pltpu.VMEM((128, C), jnp.float32),
            pltpu.SemaphoreType.DMA,
        ],
    )
    def k(hs_hbm, src_hbm, dst_hbm, out_hbm, acc, zbuf, sidx, didx, rows, sem):
        c = lax.axis_index("c")
        s = lax.axis_index("s")
        zero16 = jnp.zeros((16,), jnp.float32)

        def zb(i, _):
            zbuf[i // 8, pl.ds((i % 8) * 16, 16)] = zero16
            return 0

        lax.fori_loop(0, 1024, zb, 0)
        for j in range(5):
            pltpu.sync_copy(zbuf, acc.at[pl.ds(s * 640 + j * 128, 128)])
        plsc.subcore_barrier()

        start = c * (ER // 2) + (s * (ER // 2)) // 16
        end = c * (ER // 2) + ((s + 1) * (ER // 2)) // 16

        def body(r, _):
            pltpu.sync_copy(src_hbm.at[r], sidx)
            pltpu.sync_copy(dst_hbm.at[r], didx)
            pltpu.async_copy(hs_hbm.at[sidx], rows, sem).wait()
            pltpu.sync_copy(rows, acc.at[didx], add=True)
            return 0

        lax.fori_loop(start, end, body, 0)
        plsc.subcore_barrier()
        pltpu.sync_copy(acc.at[pl.ds(s * 640, 640)],
                        out_hbm.at[c, pl.ds(s * 640, 640)])

    return k(hs, src2d, dst2d)


def _dinv_of(d_ref):
    return lax.rsqrt(1.0 + d_ref[0, :, 0:1] + d_ref[1, :, 0:1])


def _tc_a(x_pad, W1, degp):
    def body(x_ref, w_ref, d_ref, o_ref):
        dinv = _dinv_of(d_ref)
        h = jnp.dot(x_ref[...], w_ref[...], preferred_element_type=jnp.float32)
        o_ref[...] = h * dinv

    return pl.pallas_call(
        body,
        grid=(NP // RB,),
        in_specs=[
            pl.BlockSpec((RB, C), lambda i: (i, 0)),
            pl.BlockSpec((C, C), lambda i: (0, 0)),
            pl.BlockSpec((2, RB, C), lambda i: (0, i, 0)),
        ],
        out_specs=pl.BlockSpec((RB, C), lambda i: (i, 0)),
        out_shape=jax.ShapeDtypeStruct((NP, C), jnp.float32),
    )(x_pad, W1, degp)


def _tc_b(agg, h1s, degp, b1, W2):
    def body(a_ref, h_ref, d_ref, b_ref, w_ref, o_ref):
        dinv = _dinv_of(d_ref)
        t = (a_ref[0] + a_ref[1] + h_ref[...]) * dinv + b_ref[...]
        t = jnp.maximum(t, 0.0)
        o_ref[...] = jnp.dot(t, w_ref[...],
                             preferred_element_type=jnp.float32) * dinv

    return pl.pallas_call(
        body,
        grid=(NP // RB,),
        in_specs=[
            pl.BlockSpec((2, RB, C), lambda i: (0, i, 0)),
            pl.BlockSpec((RB, C), lambda i: (i, 0)),
            pl.BlockSpec((2, RB, C), lambda i: (0, i, 0)),
            pl.BlockSpec((1, C), lambda i: (0, 0)),
            pl.BlockSpec((C, C), lambda i: (0, 0)),
        ],
        out_specs=pl.BlockSpec((RB, C), lambda i: (i, 0)),
        out_shape=jax.ShapeDtypeStruct((NP, C), jnp.float32),
    )(agg, h1s, degp, b1, W2)


def _tc_c(agg, h2s, degp, b2):
    def body(a_ref, h_ref, d_ref, b_ref, o_ref):
        dinv = _dinv_of(d_ref)
        o_ref[...] = (a_ref[0] + a_ref[1] + h_ref[...]) * dinv + b_ref[...]

    return pl.pallas_call(
        body,
        grid=(NP // RB,),
        in_specs=[
            pl.BlockSpec((2, RB, C), lambda i: (0, i, 0)),
            pl.BlockSpec((RB, C), lambda i: (i, 0)),
            pl.BlockSpec((2, RB, C), lambda i: (0, i, 0)),
            pl.BlockSpec((1, C), lambda i: (0, 0)),
        ],
        out_specs=pl.BlockSpec((RB, C), lambda i: (i, 0)),
        out_shape=jax.ShapeDtypeStruct((NP, C), jnp.float32),
    )(agg, h2s, degp, b2)


def kernel(x, edge_index, W1, b1, W2, b2):
    ei = edge_index.astype(jnp.int32)
    src2d = ei[0].reshape(ER, 128)
    dst2d = ei[1].reshape(ER, 128)
    x_pad = jnp.pad(x, ((0, NP - N), (0, 0)))

    degp = _deg_kernel(dst2d)
    h1s = _tc_a(x_pad, W1, degp)
    agg1 = _agg_kernel(h1s, src2d, dst2d)
    h2s = _tc_b(agg1, h1s, degp, b1.reshape(1, C), W2)
    agg2 = _agg_kernel(h2s, src2d, dst2d)
    outp = _tc_c(agg2, h2s, degp, b2.reshape(1, C))
    return outp[:N]


# speedup vs baseline: 27.5938x; 1.8462x over previous
import functools

import jax
import jax.numpy as jnp
from jax import lax
from jax.experimental import pallas as pl
from jax.experimental.pallas import tpu as pltpu
from jax.experimental.pallas import tpu_sc as plsc

N = 10000
NP = 10240
E = 320000
ER = E // 128
C = 128
RB = 1024


def _mesh():
    return plsc.VectorSubcoreMesh(core_axis_name="c", subcore_axis_name="s")


ROWS_T = 78
CH = 3
NCH = ROWS_T // CH
NPAIR = NCH // 2 - 1


def _deg_kernel(dst2d):
    @functools.partial(
        pl.kernel,
        mesh=_mesh(),
        out_type=jax.ShapeDtypeStruct((2, NP, C), jnp.float32),
        scratch_types=[
            pltpu.VMEM_SHARED((NP, C), jnp.float32),
            pltpu.VMEM((128, C), jnp.float32),
            pltpu.VMEM((80, 1, 128), jnp.int32),
            pltpu.SemaphoreType.DMA,
            pltpu.SemaphoreType.DMA,
        ],
    )
    def k(dst_hbm, out_hbm, acc, buf, didx, sem0, sem1):
        c = lax.axis_index("c")
        s = lax.axis_index("s")
        sems = (sem0, sem1)
        zero16 = jnp.zeros((16,), jnp.float32)
        ones16 = jnp.ones((16,), jnp.float32)

        def zb(i, _):
            buf[i // 8, pl.ds((i % 8) * 16, 16)] = zero16
            return 0

        lax.fori_loop(0, 1024, zb, 0)
        for j in range(5):
            pltpu.sync_copy(buf, acc.at[pl.ds(s * 640 + j * 128, 128)])

        def ob(i, _):
            buf[i // 8, pl.ds((i % 8) * 16, 16)] = ones16
            return 0

        lax.fori_loop(0, 1024, ob, 0)
        base = c * (ER // 2) + s * ROWS_T
        pltpu.sync_copy(dst_hbm.at[pl.ds(base, ROWS_T)],
                        didx.at[pl.ds(0, ROWS_T)])

        @pl.when(s < 2)
        def _():
            pltpu.sync_copy(dst_hbm.at[c * (ER // 2) + 16 * ROWS_T + s],
                            didx.at[ROWS_T])

        plsc.subcore_barrier()

        def scat(j, b):
            for kk in range(CH):
                pltpu.async_copy(buf, acc.at[didx.at[j * CH + kk, 0]],
                                 sems[b], add=True)

        def wait_s(b):
            for kk in range(CH):
                pltpu.make_async_copy(out_hbm.at[0, pl.ds(0, 128)], buf,
                                      sems[b]).wait()

        scat(0, 0)
        scat(1, 1)

        def pair(j2, _):
            j = 2 * j2
            wait_s(0)
            scat(j + 2, 0)
            wait_s(1)
            scat(j + 3, 1)
            return 0

        lax.fori_loop(0, NPAIR, pair, 0)
        wait_s(0)
        wait_s(1)

        @pl.when(s < 2)
        def _():
            pltpu.async_copy(buf, acc.at[didx.at[ROWS_T, 0]], sem0, add=True)
            pltpu.make_async_copy(out_hbm.at[0, pl.ds(0, 128)], buf,
                                  sem0).wait()

        plsc.subcore_barrier()
        pltpu.sync_copy(acc.at[pl.ds(s * 640, 640)],
                        out_hbm.at[c, pl.ds(s * 640, 640)])

    return k(dst2d)


def _agg_kernel(hs, src2d, dst2d):
    @functools.partial(
        pl.kernel,
        mesh=_mesh(),
        out_type=jax.ShapeDtypeStruct((2, NP, C), jnp.float32),
        scratch_types=[
            pltpu.VMEM_SHARED((NP, C), jnp.float32),
            pltpu.VMEM((128, C), jnp.float32),
            pltpu.VMEM((128, C), jnp.float32),
            pltpu.VMEM((40, 1, 128), jnp.int32),
            pltpu.VMEM((40, 1, 128), jnp.int32),
            pltpu.SemaphoreType.DMA,
            pltpu.SemaphoreType.DMA,
            pltpu.SemaphoreType.DMA,
            pltpu.SemaphoreType.DMA,
        ],
    )
    def k(hs_hbm, src_hbm, dst_hbm, out_hbm, acc, rows0, rows1,
          sidx, didx, sg0, sg1, ss0, ss1):
        c = lax.axis_index("c")
        s = lax.axis_index("s")
        rows = (rows0, rows1)
        sg = (sg0, sg1)
        ss = (ss0, ss1)
        zero16 = jnp.zeros((16,), jnp.float32)

        def zb(i, _):
            rows0[i // 8, pl.ds((i % 8) * 16, 16)] = zero16
            return 0

        lax.fori_loop(0, 1024, zb, 0)
        for j in range(5):
            pltpu.sync_copy(rows0, acc.at[pl.ds(s * 640 + j * 128, 128)])
        plsc.subcore_barrier()

        base = c * (ER // 2) + s * ROWS_T

        def gath(j, b):
            pltpu.async_copy(hs_hbm.at[sidx.at[j, 0]], rows[b], sg[b])

        def scat(j, b):
            pltpu.async_copy(rows[b], acc.at[didx.at[j, 0]], ss[b], add=True)

        def wait_g(b):
            pltpu.make_async_copy(hs_hbm.at[pl.ds(0, 128)], rows[b],
                                  sg[b]).wait()

        def wait_s(b):
            pltpu.make_async_copy(hs_hbm.at[pl.ds(0, 128)], rows[b],
                                  ss[b]).wait()

        def phase(row_base, nrows):
            pltpu.sync_copy(src_hbm.at[pl.ds(base + row_base, nrows)],
                            sidx.at[pl.ds(0, nrows)])
            pltpu.sync_copy(dst_hbm.at[pl.ds(base + row_base, nrows)],
                            didx.at[pl.ds(0, nrows)])
            gath(0, 0)
            gath(1, 1)

            def pair(j2, _):
                j = 2 * j2
                wait_g(0)
                scat(j, 0)
                wait_s(0)
                gath(j + 2, 0)
                wait_g(1)
                scat(j + 1, 1)
                wait_s(1)
                gath(j + 3, 1)
                return 0

            lax.fori_loop(0, nrows // 2 - 1, pair, 0)
            wait_g(0)
            scat(nrows - 2, 0)
            wait_g(1)
            scat(nrows - 1, 1)
            wait_s(0)
            wait_s(1)

        phase(0, 40)
        phase(40, 38)

        @pl.when(s < 2)
        def _():
            xr = c * (ER // 2) + 16 * ROWS_T + s
            pltpu.sync_copy(src_hbm.at[xr], sidx.at[0])
            pltpu.sync_copy(dst_hbm.at[xr], didx.at[0])
            gath(0, 0)
            wait_g(0)
            scat(0, 0)
            wait_s(0)

        plsc.subcore_barrier()
        pltpu.sync_copy(acc.at[pl.ds(s * 640, 640)],
                        out_hbm.at[c, pl.ds(s * 640, 640)])

    return k(hs, src2d, dst2d)


def _dinv_of(d_ref):
    return lax.rsqrt(1.0 + d_ref[0, :, 0:1] + d_ref[1, :, 0:1])


def _tc_a(x_pad, W1, degp):
    def body(x_ref, w_ref, d_ref, o_ref):
        dinv = _dinv_of(d_ref)
        h = jnp.dot(x_ref[...], w_ref[...], preferred_element_type=jnp.float32)
        o_ref[...] = h * dinv

    return pl.pallas_call(
        body,
        grid=(NP // RB,),
        in_specs=[
            pl.BlockSpec((RB, C), lambda i: (i, 0)),
            pl.BlockSpec((C, C), lambda i: (0, 0)),
            pl.BlockSpec((2, RB, C), lambda i: (0, i, 0)),
        ],
        out_specs=pl.BlockSpec((RB, C), lambda i: (i, 0)),
        out_shape=jax.ShapeDtypeStruct((NP, C), jnp.float32),
    )(x_pad, W1, degp)


def _tc_b(agg, h1s, degp, b1, W2):
    def body(a_ref, h_ref, d_ref, b_ref, w_ref, o_ref):
        dinv = _dinv_of(d_ref)
        t = (a_ref[0] + a_ref[1] + h_ref[...]) * dinv + b_ref[...]
        t = jnp.maximum(t, 0.0)
        o_ref[...] = jnp.dot(t, w_ref[...],
                             preferred_element_type=jnp.float32) * dinv

    return pl.pallas_call(
        body,
        grid=(NP // RB,),
        in_specs=[
            pl.BlockSpec((2, RB, C), lambda i: (0, i, 0)),
            pl.BlockSpec((RB, C), lambda i: (i, 0)),
            pl.BlockSpec((2, RB, C), lambda i: (0, i, 0)),
            pl.BlockSpec((1, C), lambda i: (0, 0)),
            pl.BlockSpec((C, C), lambda i: (0, 0)),
        ],
        out_specs=pl.BlockSpec((RB, C), lambda i: (i, 0)),
        out_shape=jax.ShapeDtypeStruct((NP, C), jnp.float32),
    )(agg, h1s, degp, b1, W2)


def _tc_c(agg, h2s, degp, b2):
    def body(a_ref, h_ref, d_ref, b_ref, o_ref):
        dinv = _dinv_of(d_ref)
        o_ref[...] = (a_ref[0] + a_ref[1] + h_ref[...]) * dinv + b_ref[...]

    return pl.pallas_call(
        body,
        grid=(NP // RB,),
        in_specs=[
            pl.BlockSpec((2, RB, C), lambda i: (0, i, 0)),
            pl.BlockSpec((RB, C), lambda i: (i, 0)),
            pl.BlockSpec((2, RB, C), lambda i: (0, i, 0)),
            pl.BlockSpec((1, C), lambda i: (0, 0)),
        ],
        out_specs=pl.BlockSpec((RB, C), lambda i: (i, 0)),
        out_shape=jax.ShapeDtypeStruct((NP, C), jnp.float32),
    )(agg, h2s, degp, b2)


def kernel(x, edge_index, W1, b1, W2, b2):
    ei = edge_index.astype(jnp.int32)
    src2d = ei[0].reshape(ER, 1, 128)
    dst2d = ei[1].reshape(ER, 1, 128)
    x_pad = jnp.pad(x, ((0, NP - N), (0, 0)))

    degp = _deg_kernel(dst2d)
    h1s = _tc_a(x_pad, W1, degp)
    agg1 = _agg_kernel(h1s, src2d, dst2d)
    h2s = _tc_b(agg1, h1s, degp, b1.reshape(1, C), W2)
    agg2 = _agg_kernel(h2s, src2d, dst2d)
    outp = _tc_c(agg2, h2s, degp, b2.reshape(1, C))
    return outp[:N]
